# Initial kernel scaffold; baseline (speedup 1.0000x reference)
import jax
import jax.numpy as jnp
from jax.experimental import pallas as pl


def kernel(x, mu, std):
    raise NotImplementedError("write your pallas kernel here")


# speedup vs baseline: 3.1581x; 3.1581x over previous
import jax
import jax.numpy as jnp
from jax.experimental import pallas as pl
from jax.experimental.pallas import tpu as pltpu

_BN = 1024


def _gauss_body(std_row_ref, std_col_ref, mu_t_ref, x_ref, out_ref):
    inv_row = 1.0 / std_row_ref[...]
    inv_col = 1.0 / std_col_ref[...]
    mu_t = mu_t_ref[...]
    muw_t = mu_t * inv_col
    msq_half = 0.5 * jnp.sum(mu_t * muw_t, axis=0, keepdims=True)
    xb = x_ref[...]
    xsq_half = 0.5 * jnp.sum(xb * xb * inv_row, axis=1, keepdims=True)
    cross = jnp.dot(xb, muw_t, preferred_element_type=jnp.float32)
    out_ref[...] = jnp.exp(cross - xsq_half - msq_half)


def kernel(x, mu, std):
    n, d = x.shape
    k = mu.shape[0]
    mu_t = mu[:, 0, :].T
    std_row = std.reshape(1, d)
    std_col = std.reshape(d, 1)
    return pl.pallas_call(
        _gauss_body,
        grid=(n // _BN,),
        in_specs=[
            pl.BlockSpec((1, d), lambda i: (0, 0)),
            pl.BlockSpec((d, 1), lambda i: (0, 0)),
            pl.BlockSpec((d, k), lambda i: (0, 0)),
            pl.BlockSpec((_BN, d), lambda i: (i, 0)),
        ],
        out_specs=pl.BlockSpec((_BN, k), lambda i: (i, 0)),
        out_shape=jax.ShapeDtypeStruct((n, k), jnp.float32),
        compiler_params=pltpu.CompilerParams(
            dimension_semantics=("parallel",),
            vmem_limit_bytes=60 * 1024 * 1024,
        ),
    )(std_row, std_col, mu_t, x)
